# Initial kernel scaffold; baseline (speedup 1.0000x reference)
import jax
import jax.numpy as jnp
from jax.experimental import pallas as pl


def kernel(radar_points, original_image_size):
    raise NotImplementedError("write your pallas kernel here")


# speedup vs baseline: 8.3904x; 8.3904x over previous
import functools

import jax
import jax.numpy as jnp
from jax import lax
from jax.experimental import pallas as pl
from jax.experimental.pallas import tpu as pltpu
from jax.experimental.pallas import tpu_sc as plsc

H_TGT = 680
W_TGT = 680
NBINS = H_TGT * W_TGT
NSUB = 16
LANES = 16
CHUNK = 128
C_ROWS = 49
TILE_PTS = C_ROWS * CHUNK
NBINS_PAD = 462848
STRIPE = NBINS_PAD // NSUB
TRASH0 = NBINS


def _round_half_even(x):
    f = x.astype(jnp.int32)
    r = x - f.astype(jnp.float32)
    tie_up = (r == 0.5) & ((f & 1) == 1)
    up = (r > 0.5) | tie_up
    return f + up.astype(jnp.int32)


def _sc_histogram(cols, scales, zeros_seg, ones_tile, n_valid, n_tiles):
    pts_per_sub = n_tiles * TILE_PTS
    n_pad = NSUB * pts_per_sub
    mesh = plsc.VectorSubcoreMesh(core_axis_name="c", subcore_axis_name="s")

    @functools.partial(
        pl.kernel,
        out_type=jax.ShapeDtypeStruct((5 * NBINS_PAD,), jnp.float32),
        mesh=mesh,
        compiler_params=pltpu.CompilerParams(needs_layout_passes=False),
        scratch_types=[
            pltpu.VMEM((TILE_PTS,), jnp.float32),
            pltpu.VMEM((TILE_PTS,), jnp.float32),
            pltpu.VMEM((TILE_PTS,), jnp.float32),
            pltpu.VMEM((TILE_PTS,), jnp.float32),
            pltpu.VMEM((TILE_PTS,), jnp.int32),
            pltpu.VMEM((TILE_PTS,), jnp.float32),
            pltpu.VMEM((2, LANES), jnp.float32),
            pltpu.VMEM_SHARED((NBINS_PAD,), jnp.float32),
            pltpu.VMEM_SHARED((NBINS_PAD,), jnp.float32),
            pltpu.VMEM_SHARED((NBINS_PAD,), jnp.float32),
        ],
    )
    def hist_kernel(cols_hbm, scales_hbm, zeros_hbm, ones_hbm, out_hbm,
                    u_v, v_v, f0_v, f1_v, idx_v, ones_v, scal_v,
                    hist_a, hist_b, hist_c):
        c = lax.axis_index("c")
        s = lax.axis_index("s")

        stripe = pl.ds(s * STRIPE, STRIPE)
        pltpu.sync_copy(zeros_hbm, hist_a.at[stripe])
        pltpu.sync_copy(zeros_hbm, hist_b.at[stripe])
        pltpu.sync_copy(zeros_hbm, hist_c.at[stripe])
        pltpu.sync_copy(ones_hbm, ones_v)
        pltpu.sync_copy(scales_hbm, scal_v)
        plsc.subcore_barrier()

        ws = scal_v[0, :]
        hs = scal_v[1, :]
        lane = lax.iota(jnp.int32, LANES)
        trash = TRASH0 + s * LANES + lane
        ch_f0 = 2 + 2 * c
        ch_f1 = 3 + 2 * c

        @pl.loop(0, n_tiles)
        def _tile(t):
            tile_base = s * pts_per_sub + t * TILE_PTS
            pltpu.sync_copy(cols_hbm.at[pl.ds(tile_base, TILE_PTS)], u_v)
            pltpu.sync_copy(cols_hbm.at[pl.ds(n_pad + tile_base, TILE_PTS)],
                            v_v)
            pltpu.sync_copy(cols_hbm.at[pl.ds(ch_f0 * n_pad + tile_base,
                                              TILE_PTS)], f0_v)
            pltpu.sync_copy(cols_hbm.at[pl.ds(ch_f1 * n_pad + tile_base,
                                              TILE_PTS)], f1_v)

            @pl.loop(0, TILE_PTS // LANES)
            def _vec(k):
                sl = pl.ds(k * LANES, LANES)
                ui = _round_half_even(u_v[sl] * ws)
                vi = _round_half_even(v_v[sl] * hs)
                ui = jnp.minimum(jnp.maximum(ui, 0), W_TGT - 1)
                vi = jnp.minimum(jnp.maximum(vi, 0), H_TGT - 1)
                flat = vi * W_TGT + ui
                pos = tile_base + k * LANES + lane
                flat = jnp.where(pos < n_valid, flat, trash)
                idx_v[sl] = flat

            @pl.when(c == 0)
            def _():
                pltpu.sync_copy(ones_v, hist_a.at[idx_v], add=True)
            pltpu.sync_copy(f0_v, hist_b.at[idx_v], add=True)
            pltpu.sync_copy(f1_v, hist_c.at[idx_v], add=True)

        plsc.subcore_barrier()

        @pl.when(c == 0)
        def _():
            pltpu.sync_copy(hist_a.at[stripe],
                            out_hbm.at[pl.ds(s * STRIPE, STRIPE)])
            pltpu.sync_copy(hist_b.at[stripe],
                            out_hbm.at[pl.ds(NBINS_PAD + s * STRIPE, STRIPE)])
            pltpu.sync_copy(hist_c.at[stripe],
                            out_hbm.at[pl.ds(2 * NBINS_PAD + s * STRIPE,
                                             STRIPE)])

        @pl.when(c == 1)
        def _():
            pltpu.sync_copy(hist_b.at[stripe],
                            out_hbm.at[pl.ds(3 * NBINS_PAD + s * STRIPE,
                                             STRIPE)])
            pltpu.sync_copy(hist_c.at[stripe],
                            out_hbm.at[pl.ds(4 * NBINS_PAD + s * STRIPE,
                                             STRIPE)])

    return hist_kernel(cols, scales, zeros_seg, ones_tile)


def _finalize_body(h_ref, o_ref):
    cts = h_ref[0]
    zero = cts == 0.0
    safe = jnp.where(zero, 1.0, cts)
    o_ref[0] = jnp.where(zero, 0.0, h_ref[1] / safe)
    o_ref[1] = jnp.where(zero, 0.0, h_ref[2] / safe)
    o_ref[2] = jnp.where(zero, 0.0, h_ref[3] / safe)
    o_ref[3] = jnp.where(zero, 0.0, h_ref[4] / safe)


def kernel(radar_points, original_image_size):
    n = radar_points.shape[0]
    n_tiles = -(-n // (NSUB * TILE_PTS))
    n_pad = NSUB * n_tiles * TILE_PTS

    h_orig = original_image_size[0].astype(jnp.float32)
    w_orig = original_image_size[1].astype(jnp.float32)
    w_scale = W_TGT / w_orig
    h_scale = H_TGT / h_orig

    cols = jnp.pad(radar_points.T, ((0, 0), (0, n_pad - n))).reshape(-1)
    scales = jnp.stack([jnp.full((LANES,), w_scale, jnp.float32),
                        jnp.full((LANES,), h_scale, jnp.float32)])
    zeros_seg = jnp.zeros((STRIPE,), jnp.float32)
    ones_tile = jnp.ones((TILE_PTS,), jnp.float32)

    hists = _sc_histogram(cols, scales, zeros_seg, ones_tile, n, n_tiles)

    rows = NBINS_PAD // CHUNK
    blk = 32
    img = pl.pallas_call(
        _finalize_body,
        grid=(rows // blk,),
        in_specs=[pl.BlockSpec((5, blk, CHUNK), lambda i: (0, i, 0))],
        out_specs=pl.BlockSpec((4, blk, CHUNK), lambda i: (0, i, 0)),
        out_shape=jax.ShapeDtypeStruct((4, rows, CHUNK), jnp.float32),
    )(hists.reshape(5, rows, CHUNK))

    return img.reshape(4, NBINS_PAD)[:, :NBINS].reshape(4, H_TGT, W_TGT)


# speedup vs baseline: 20.5787x; 2.4526x over previous
import functools

import jax
import jax.numpy as jnp
from jax import lax
from jax.experimental import pallas as pl
from jax.experimental.pallas import tpu as pltpu
from jax.experimental.pallas import tpu_sc as plsc

H_TGT = 680
W_TGT = 680
NBINS = H_TGT * W_TGT
NSUB = 16
LANES = 16
CHUNK = 128
C_ROWS = 49
TILE_PTS = C_ROWS * CHUNK
NBINS_PAD = 462848
STRIPE = NBINS_PAD // NSUB
TRASH0 = NBINS


def _idx_body(n_blk, n_valid, uv_ref, s_ref, o_ref):
    j = pl.program_id(0)
    ws = s_ref[0, 0]
    hs = s_ref[1, 0]
    u = uv_ref[0:1, :]
    v = uv_ref[1:2, :]
    ui = jnp.clip(jnp.round(u * ws).astype(jnp.int32), 0, W_TGT - 1)
    vi = jnp.clip(jnp.round(v * hs).astype(jnp.int32), 0, H_TGT - 1)
    flat = vi * W_TGT + ui
    pos = j * n_blk + lax.broadcasted_iota(jnp.int32, flat.shape, 1)
    o_ref[...] = jnp.where(pos < n_valid, flat, TRASH0 + (pos & 255))


def _sc_histogram(feats, idx, zeros_seg, ones_tile, n_tiles):
    pts_per_sub = n_tiles * TILE_PTS
    n_pad = NSUB * pts_per_sub
    mesh = plsc.VectorSubcoreMesh(core_axis_name="c", subcore_axis_name="s")

    @functools.partial(
        pl.kernel,
        out_type=jax.ShapeDtypeStruct((5 * NBINS_PAD,), jnp.float32),
        mesh=mesh,
        compiler_params=pltpu.CompilerParams(needs_layout_passes=False),
        scratch_types=[
            pltpu.VMEM((TILE_PTS,), jnp.float32),
            pltpu.VMEM((TILE_PTS,), jnp.float32),
            pltpu.VMEM((TILE_PTS,), jnp.int32),
            pltpu.VMEM((TILE_PTS,), jnp.float32),
            pltpu.VMEM_SHARED((NBINS_PAD,), jnp.float32),
            pltpu.VMEM_SHARED((NBINS_PAD,), jnp.float32),
            pltpu.VMEM_SHARED((NBINS_PAD,), jnp.float32),
        ],
    )
    def hist_kernel(feats_hbm, idx_hbm, zeros_hbm, ones_hbm, out_hbm,
                    f0_v, f1_v, idx_v, ones_v, hist_a, hist_b, hist_c):
        c = lax.axis_index("c")
        s = lax.axis_index("s")

        stripe = pl.ds(s * STRIPE, STRIPE)
        pltpu.sync_copy(zeros_hbm, hist_a.at[stripe])
        pltpu.sync_copy(zeros_hbm, hist_b.at[stripe])
        pltpu.sync_copy(zeros_hbm, hist_c.at[stripe])
        pltpu.sync_copy(ones_hbm, ones_v)
        plsc.subcore_barrier()

        ch_f0 = 2 * c
        ch_f1 = 2 * c + 1

        @pl.loop(0, n_tiles)
        def _tile(t):
            tile_base = s * pts_per_sub + t * TILE_PTS
            pltpu.sync_copy(idx_hbm.at[pl.ds(tile_base, TILE_PTS)], idx_v)
            pltpu.sync_copy(feats_hbm.at[pl.ds(ch_f0 * n_pad + tile_base,
                                               TILE_PTS)], f0_v)
            pltpu.sync_copy(feats_hbm.at[pl.ds(ch_f1 * n_pad + tile_base,
                                               TILE_PTS)], f1_v)

            @pl.when(c == 0)
            def _():
                pltpu.sync_copy(ones_v, hist_a.at[idx_v], add=True)
            pltpu.sync_copy(f0_v, hist_b.at[idx_v], add=True)
            pltpu.sync_copy(f1_v, hist_c.at[idx_v], add=True)

        plsc.subcore_barrier()

        @pl.when(c == 0)
        def _():
            pltpu.sync_copy(hist_a.at[stripe],
                            out_hbm.at[pl.ds(s * STRIPE, STRIPE)])
            pltpu.sync_copy(hist_b.at[stripe],
                            out_hbm.at[pl.ds(NBINS_PAD + s * STRIPE, STRIPE)])
            pltpu.sync_copy(hist_c.at[stripe],
                            out_hbm.at[pl.ds(2 * NBINS_PAD + s * STRIPE,
                                             STRIPE)])

        @pl.when(c == 1)
        def _():
            pltpu.sync_copy(hist_b.at[stripe],
                            out_hbm.at[pl.ds(3 * NBINS_PAD + s * STRIPE,
                                             STRIPE)])
            pltpu.sync_copy(hist_c.at[stripe],
                            out_hbm.at[pl.ds(4 * NBINS_PAD + s * STRIPE,
                                             STRIPE)])

    return hist_kernel(feats, idx, zeros_seg, ones_tile)


def _finalize_body(h_ref, o_ref):
    cts = h_ref[0]
    zero = cts == 0.0
    safe = jnp.where(zero, 1.0, cts)
    o_ref[0] = jnp.where(zero, 0.0, h_ref[1] / safe)
    o_ref[1] = jnp.where(zero, 0.0, h_ref[2] / safe)
    o_ref[2] = jnp.where(zero, 0.0, h_ref[3] / safe)
    o_ref[3] = jnp.where(zero, 0.0, h_ref[4] / safe)


def kernel(radar_points, original_image_size):
    n = radar_points.shape[0]
    n_tiles = -(-n // (NSUB * TILE_PTS))
    n_pad = NSUB * n_tiles * TILE_PTS

    h_orig = original_image_size[0].astype(jnp.float32)
    w_orig = original_image_size[1].astype(jnp.float32)
    w_scale = W_TGT / w_orig
    h_scale = H_TGT / h_orig

    cols = jnp.pad(radar_points.T, ((0, 0), (0, n_pad - n)))
    scales = jnp.stack([w_scale, h_scale]).reshape(2, 1)

    n_blk = n_pad // 10
    idx = pl.pallas_call(
        functools.partial(_idx_body, n_blk, n),
        grid=(n_pad // n_blk,),
        in_specs=[pl.BlockSpec((2, n_blk), lambda j: (0, j)),
                  pl.BlockSpec((2, 1), lambda j: (0, 0))],
        out_specs=pl.BlockSpec((1, n_blk), lambda j: (0, j)),
        out_shape=jax.ShapeDtypeStruct((1, n_pad), jnp.int32),
    )(cols[:2], scales).reshape(n_pad)

    feats = cols[2:].reshape(-1)
    zeros_seg = jnp.zeros((STRIPE,), jnp.float32)
    ones_tile = jnp.ones((TILE_PTS,), jnp.float32)

    hists = _sc_histogram(feats, idx, zeros_seg, ones_tile, n_tiles)

    rows = NBINS_PAD // CHUNK
    blk = 32
    img = pl.pallas_call(
        _finalize_body,
        grid=(rows // blk,),
        in_specs=[pl.BlockSpec((5, blk, CHUNK), lambda i: (0, i, 0))],
        out_specs=pl.BlockSpec((4, blk, CHUNK), lambda i: (0, i, 0)),
        out_shape=jax.ShapeDtypeStruct((4, rows, CHUNK), jnp.float32),
    )(hists.reshape(5, rows, CHUNK))

    return img.reshape(4, NBINS_PAD)[:, :NBINS].reshape(4, H_TGT, W_TGT)


# speedup vs baseline: 23.1891x; 1.1269x over previous
import functools

import jax
import jax.numpy as jnp
from jax import lax
from jax.experimental import pallas as pl
from jax.experimental.pallas import tpu as pltpu
from jax.experimental.pallas import tpu_sc as plsc

H_TGT = 680
W_TGT = 680
NBINS = H_TGT * W_TGT
NSUB = 16
CHUNK = 128
C_ROWS = 49
TILE_PTS = C_ROWS * CHUNK
NBINS_PAD = 462848
STRIPE = NBINS_PAD // NSUB
TRASH0 = NBINS
NCH_OUT = 6


def _idx_body(n_blk, n_valid, uv_ref, s_ref, o_ref):
    j = pl.program_id(0)
    ws = s_ref[0, 0]
    hs = s_ref[1, 0]
    u = uv_ref[0:1, :]
    v = uv_ref[1:2, :]
    ui = jnp.clip(jnp.round(u * ws).astype(jnp.int32), 0, W_TGT - 1)
    vi = jnp.clip(jnp.round(v * hs).astype(jnp.int32), 0, H_TGT - 1)
    flat = vi * W_TGT + ui
    pos = j * n_blk + lax.broadcasted_iota(jnp.int32, flat.shape, 1)
    o_ref[...] = jnp.where(pos < n_valid, flat, TRASH0 + (pos & 255))


def _sc_histogram(feats, idx, zeros_seg, ones_tile, n_tiles):
    pts_per_sub = n_tiles * TILE_PTS
    n_pad = NSUB * pts_per_sub
    half = n_tiles // 2
    mesh = plsc.VectorSubcoreMesh(core_axis_name="c", subcore_axis_name="s")

    vmem_f = pltpu.VMEM((TILE_PTS,), jnp.float32)
    vmem_i = pltpu.VMEM((TILE_PTS,), jnp.int32)

    @functools.partial(
        pl.kernel,
        out_type=jax.ShapeDtypeStruct((NCH_OUT * NBINS_PAD,), jnp.float32),
        mesh=mesh,
        compiler_params=pltpu.CompilerParams(needs_layout_passes=False),
        scratch_types=[
            vmem_f, vmem_f, vmem_i,
            vmem_f, vmem_f, vmem_i,
            vmem_f,
            pltpu.VMEM_SHARED((NBINS_PAD,), jnp.float32),
            pltpu.VMEM_SHARED((NBINS_PAD,), jnp.float32),
            pltpu.VMEM_SHARED((NBINS_PAD,), jnp.float32),
            pltpu.SemaphoreType.DMA,
            pltpu.SemaphoreType.DMA,
            pltpu.SemaphoreType.DMA,
            pltpu.SemaphoreType.DMA,
            pltpu.SemaphoreType.DMA,
            pltpu.SemaphoreType.DMA,
        ],
    )
    def hist_kernel(feats_hbm, idx_hbm, zeros_hbm, ones_hbm, out_hbm,
                    f0_0, f1_0, idx_0,
                    f0_1, f1_1, idx_1,
                    ones_v, hist_a, hist_b, hist_c,
                    sa_0, sb_0, sc_0, sa_1, sb_1, sc_1):
        c = lax.axis_index("c")
        s = lax.axis_index("s")
        f0s, f1s, idxs = [f0_0, f0_1], [f1_0, f1_1], [idx_0, idx_1]
        sas, sbs, scs = [sa_0, sa_1], [sb_0, sb_1], [sc_0, sc_1]

        stripe = pl.ds(s * STRIPE, STRIPE)
        pltpu.sync_copy(zeros_hbm, hist_a.at[stripe])
        pltpu.sync_copy(zeros_hbm, hist_b.at[stripe])
        pltpu.sync_copy(zeros_hbm, hist_c.at[stripe])
        pltpu.sync_copy(ones_hbm, ones_v)
        plsc.subcore_barrier()

        ch_f0 = 2 * c
        ch_f1 = 2 * c + 1

        pending = {}

        def stage(t):
            b = t % 2
            base = s * pts_per_sub + t * TILE_PTS
            pltpu.sync_copy(idx_hbm.at[pl.ds(base, TILE_PTS)], idxs[b])
            pltpu.sync_copy(feats_hbm.at[pl.ds(ch_f0 * n_pad + base,
                                               TILE_PTS)], f0s[b])
            pltpu.sync_copy(feats_hbm.at[pl.ds(ch_f1 * n_pad + base,
                                               TILE_PTS)], f1s[b])

        def fire(t):
            b = t % 2
            descs = []
            count_core = "c0" if t < half else "c1"

            @pl.when(c == (0 if t < half else 1))
            def _():
                descs.append((count_core, pltpu.async_copy(
                    ones_v, hist_a.at[idxs[b]], sas[b], add=True)))

            descs.append((None, pltpu.async_copy(
                f0s[b], hist_b.at[idxs[b]], sbs[b], add=True)))
            descs.append((None, pltpu.async_copy(
                f1s[b], hist_c.at[idxs[b]], scs[b], add=True)))
            pending[b] = descs

        def drain(b):
            for cond, d in pending.get(b, []):
                if cond is None:
                    d.wait()
                elif cond == "c0":
                    @pl.when(c == 0)
                    def _():
                        d.wait()
                else:
                    @pl.when(c == 1)
                    def _():
                        d.wait()
            pending[b] = []

        for t in range(n_tiles):
            drain(t % 2)
            stage(t)
            fire(t)
        drain(0)
        drain(1)

        plsc.subcore_barrier()

        @pl.when(c == 0)
        def _():
            pltpu.sync_copy(hist_a.at[stripe],
                            out_hbm.at[pl.ds(s * STRIPE, STRIPE)])
            pltpu.sync_copy(hist_b.at[stripe],
                            out_hbm.at[pl.ds(NBINS_PAD + s * STRIPE, STRIPE)])
            pltpu.sync_copy(hist_c.at[stripe],
                            out_hbm.at[pl.ds(2 * NBINS_PAD + s * STRIPE,
                                             STRIPE)])

        @pl.when(c == 1)
        def _():
            pltpu.sync_copy(hist_b.at[stripe],
                            out_hbm.at[pl.ds(3 * NBINS_PAD + s * STRIPE,
                                             STRIPE)])
            pltpu.sync_copy(hist_c.at[stripe],
                            out_hbm.at[pl.ds(4 * NBINS_PAD + s * STRIPE,
                                             STRIPE)])
            pltpu.sync_copy(hist_a.at[stripe],
                            out_hbm.at[pl.ds(5 * NBINS_PAD + s * STRIPE,
                                             STRIPE)])

    return hist_kernel(feats, idx, zeros_seg, ones_tile)


def _finalize_body(h_ref, o_ref):
    cts = h_ref[0] + h_ref[5]
    zero = cts == 0.0
    safe = jnp.where(zero, 1.0, cts)
    o_ref[0] = jnp.where(zero, 0.0, h_ref[1] / safe)
    o_ref[1] = jnp.where(zero, 0.0, h_ref[2] / safe)
    o_ref[2] = jnp.where(zero, 0.0, h_ref[3] / safe)
    o_ref[3] = jnp.where(zero, 0.0, h_ref[4] / safe)


def kernel(radar_points, original_image_size):
    n = radar_points.shape[0]
    n_tiles = -(-n // (NSUB * TILE_PTS))
    n_pad = NSUB * n_tiles * TILE_PTS

    h_orig = original_image_size[0].astype(jnp.float32)
    w_orig = original_image_size[1].astype(jnp.float32)
    w_scale = W_TGT / w_orig
    h_scale = H_TGT / h_orig

    cols = jnp.pad(radar_points.T, ((0, 0), (0, n_pad - n)))
    scales = jnp.stack([w_scale, h_scale]).reshape(2, 1)

    n_blk = n_pad // 10
    idx = pl.pallas_call(
        functools.partial(_idx_body, n_blk, n),
        grid=(n_pad // n_blk,),
        in_specs=[pl.BlockSpec((2, n_blk), lambda j: (0, j)),
                  pl.BlockSpec((2, 1), lambda j: (0, 0))],
        out_specs=pl.BlockSpec((1, n_blk), lambda j: (0, j)),
        out_shape=jax.ShapeDtypeStruct((1, n_pad), jnp.int32),
    )(cols[:2], scales).reshape(n_pad)

    feats = cols[2:].reshape(-1)
    zeros_seg = jnp.zeros((STRIPE,), jnp.float32)
    ones_tile = jnp.ones((TILE_PTS,), jnp.float32)

    hists = _sc_histogram(feats, idx, zeros_seg, ones_tile, n_tiles)

    rows = NBINS_PAD // CHUNK
    blk = 32
    img = pl.pallas_call(
        _finalize_body,
        grid=(rows // blk,),
        in_specs=[pl.BlockSpec((NCH_OUT, blk, CHUNK), lambda i: (0, i, 0))],
        out_specs=pl.BlockSpec((4, blk, CHUNK), lambda i: (0, i, 0)),
        out_shape=jax.ShapeDtypeStruct((4, rows, CHUNK), jnp.float32),
    )(hists.reshape(NCH_OUT, rows, CHUNK))

    return img.reshape(4, NBINS_PAD)[:, :NBINS].reshape(4, H_TGT, W_TGT)
